# baseline (speedup 1.0000x reference)
import functools

import jax
import jax.numpy as jnp
from jax import lax
from jax.experimental import pallas as pl
from jax.experimental.pallas import tpu as pltpu
from jax.experimental.pallas import tpu_sc as plsc

B = 10000
N = 100000
D = 128
H1 = 128
H2 = 64
C = 64
T = 3
S0, S1 = 5, 2

_NC, _NS = 2, 16
_NW = _NC * _NS

BP = 10240
BS = 400
G = B // BS

STEPS = [(0, 0, 0), (0, 1, 0), (1, 0, 1), (2, 0, 2), (2, 1, 1), (2, 2, 0)]


def _mk_gather(rows, nch, ch, nb):
    per_w = nch * ch
    assert rows == _NW * per_w and nch % nb == 0
    mesh = plsc.VectorSubcoreMesh(core_axis_name="c", subcore_axis_name="s")

    @functools.partial(
        pl.kernel,
        out_type=jax.ShapeDtypeStruct((rows, D), jnp.float32),
        mesh=mesh,
        scratch_types=(
            [pltpu.VMEM((nch, ch), jnp.int32)]
            + [pltpu.VMEM((ch, D), jnp.float32) for _ in range(nb)]
            + [pltpu.SemaphoreType.DMA for _ in range(nb)]
        ),
    )
    def k(x_hbm, idx_hbm, out_hbm, idx_v, *rest):
        bufs, sems = rest[:nb], rest[nb:]
        wid = lax.axis_index("s") * _NC + lax.axis_index("c")
        base = wid * per_w
        pltpu.sync_copy(idx_hbm.at[wid], idx_v)

        @pl.loop(0, nch // nb)
        def _grp(i):
            j0 = i * nb
            cps = [pltpu.async_copy(x_hbm.at[idx_v.at[j0 + b]], bufs[b],
                                    sems[b]) for b in range(nb)]
            for b in range(nb):
                cps[b].wait()
                pltpu.sync_copy(
                    bufs[b], out_hbm.at[pl.ds(base + (j0 + b) * ch, ch)])

    return k


def _mk_gather_pairsum(rows, nch, ch, nb):
    per_w = nch * ch
    assert rows == _NW * per_w and nch % nb == 0
    mesh = plsc.VectorSubcoreMesh(core_axis_name="c", subcore_axis_name="s")

    @functools.partial(
        pl.kernel,
        out_type=jax.ShapeDtypeStruct((rows, D), jnp.float32),
        mesh=mesh,
        scratch_types=(
            [pltpu.VMEM((nch, ch), jnp.int32) for _ in range(2)]
            + [pltpu.VMEM((ch, D), jnp.float32) for _ in range(nb)]
            + [pltpu.SemaphoreType.DMA for _ in range(2 * nb)]
        ),
    )
    def k(x_hbm, idxe_hbm, idxo_hbm, out_hbm, idxe_v, idxo_v, *rest):
        bufs = rest[:nb]
        esems = rest[nb:2 * nb]
        osems = rest[2 * nb:]
        wid = lax.axis_index("s") * _NC + lax.axis_index("c")
        base = wid * per_w
        pltpu.sync_copy(idxe_hbm.at[wid], idxe_v)
        pltpu.sync_copy(idxo_hbm.at[wid], idxo_v)

        @pl.loop(0, nch // nb)
        def _grp(i):
            j0 = i * nb
            ecps = [pltpu.async_copy(x_hbm.at[idxe_v.at[j0 + b]], bufs[b],
                                     esems[b]) for b in range(nb)]
            ocps = [None] * nb
            for b in range(nb):
                ecps[b].wait()
                ocps[b] = pltpu.async_copy(x_hbm.at[idxo_v.at[j0 + b]],
                                           bufs[b], osems[b], add=True)
            for b in range(nb):
                ocps[b].wait()
                pltpu.sync_copy(
                    bufs[b], out_hbm.at[pl.ds(base + (j0 + b) * ch, ch)])

    return k


_gather_h0 = _mk_gather(BP, 4, 80, 4)
_gather_h1 = _mk_gather(T * S0 * BP, 50, 96, 5)
_gather_n2 = _mk_gather_pairsum(T * S0 * BP, 50, 96, 5)


def _dense_body(h0_ref, h1_ref, n2_ref,
                w1l_ref, w1r_ref, b1_ref,
                w2l_ref, w2r_ref, b2_ref,
                pw_ref, mtg_ref, cb_ref, out_ref):
    f32 = jnp.float32
    h0 = h0_ref[...]
    acc = jnp.zeros((BS, C), f32)
    v1 = {}
    v2 = {}
    for si, (t, c, _k) in enumerate(STEPS):
        h1 = h1_ref[t]
        n1 = (h1[0] + h1[1] + h1[2] + h1[3] + h1[4]) * (1.0 / S0)
        n2 = n2_ref[t] * (1.0 / S1)
        h1f = h1.reshape(S0 * BS, D)
        n2f = n2.reshape(S0 * BS, D)
        xcat = jnp.concatenate([h0, h1f], axis=0)
        ncat = jnp.concatenate([n1, n2f], axis=0)
        u = (jnp.dot(xcat, w1l_ref[c], preferred_element_type=f32)
             + jnp.dot(ncat, w1r_ref[c], preferred_element_type=f32)
             + b1_ref[c][None, :])
        ux = u[:, :H1]
        ut = u[:, H1:]
        v = v1.get(c, 0.0) + ut
        spk = (v >= 1.0).astype(f32)
        v1[c] = v * (1.0 - spk)
        outs = jax.nn.sigmoid(ux) * spk
        h0n = outs[:BS]
        nb = outs[BS:].reshape(S0, BS, H1)
        nb = (nb[0] + nb[1] + nb[2] + nb[3] + nb[4]) * (1.0 / S0)
        u2 = (jnp.dot(h0n, w2l_ref[c], preferred_element_type=f32)
              + jnp.dot(nb, w2r_ref[c], preferred_element_type=f32)
              + b2_ref[c][None, :])
        ux2 = u2[:, :H2]
        ut2 = u2[:, H2:]
        v_ = v2.get(c, 0.0) + ut2
        spk2 = (v_ >= 1.0).astype(f32)
        v2[c] = v_ * (1.0 - spk2)
        o = jax.nn.sigmoid(ux2) * spk2
        acc = acc + o * pw_ref[si][None, :]
    out_ref[...] = jnp.dot(acc, mtg_ref[...], preferred_element_type=f32) \
        + cb_ref[...][None, :]


def _dense(h0, h1, n2s, w1l, w1r, b1, w2l, w2r, b2, pw, mtg, cb):
    return pl.pallas_call(
        _dense_body,
        grid=(G,),
        in_specs=[
            pl.BlockSpec((BS, D), lambda i: (i, 0)),
            pl.BlockSpec((T, S0, BS, D), lambda i: (0, 0, i, 0)),
            pl.BlockSpec((T, S0, BS, D), lambda i: (0, 0, i, 0)),
            pl.BlockSpec((T, D, 2 * H1), lambda i: (0, 0, 0)),
            pl.BlockSpec((T, D, 2 * H1), lambda i: (0, 0, 0)),
            pl.BlockSpec((T, 2 * H1), lambda i: (0, 0)),
            pl.BlockSpec((T, H1, 2 * H2), lambda i: (0, 0, 0)),
            pl.BlockSpec((T, H1, 2 * H2), lambda i: (0, 0, 0)),
            pl.BlockSpec((T, 2 * H2), lambda i: (0, 0)),
            pl.BlockSpec((len(STEPS), H2), lambda i: (0, 0)),
            pl.BlockSpec((H2, C), lambda i: (0, 0)),
            pl.BlockSpec((C,), lambda i: (0,)),
        ],
        out_specs=pl.BlockSpec((BS, C), lambda i: (i, 0)),
        out_shape=jax.ShapeDtypeStruct((B, C), jnp.float32),
        compiler_params=pltpu.CompilerParams(
            dimension_semantics=("arbitrary",),
        ),
    )(h0, h1, n2s, w1l, w1r, b1, w2l, w2r, b2, pw, mtg, cb)


def kernel(x, nodes, nbr1, nbr2,
           A1_Wl, A1_Wr, A1_Wlt, A1_Wrt, A1_bl, A1_br, A1_blt, A1_brt,
           A2_Wl, A2_Wr, A2_Wlt, A2_Wrt, A2_bl, A2_br, A2_blt, A2_brt,
           MTG_W, MTG_b, pw1, pb1, pw2, pb2, pw3, pb3):
    nodes = nodes.astype(jnp.int32)
    nbr1 = nbr1.astype(jnp.int32)
    nbr2 = nbr2.astype(jnp.int32)
    pad = BP - B

    idx0 = jnp.pad(nodes, (0, pad)).reshape(_NW, 4, 80)
    idx1 = nbr1.reshape(T, B, S0).transpose(0, 2, 1)
    idx1 = jnp.pad(idx1, ((0, 0), (0, 0), (0, pad))).reshape(_NW, 50, 96)
    idx2 = nbr2.reshape(T, B, S0, S1).transpose(0, 2, 1, 3)
    idxe = jnp.pad(idx2[..., 0], ((0, 0), (0, 0), (0, pad))).reshape(_NW, 50, 96)
    idxo = jnp.pad(idx2[..., 1], ((0, 0), (0, 0), (0, pad))).reshape(_NW, 50, 96)

    h0 = _gather_h0(x, idx0)
    h1 = _gather_h1(x, idx1).reshape(T, S0, BP, D)
    n2s = _gather_n2(x, idxe, idxo).reshape(T, S0, BP, D)

    w1l = jnp.concatenate([A1_Wl.transpose(0, 2, 1),
                           A1_Wlt.transpose(0, 2, 1)], axis=2)
    w1r = jnp.concatenate([A1_Wr.transpose(0, 2, 1),
                           A1_Wrt.transpose(0, 2, 1)], axis=2)
    b1 = jnp.concatenate([A1_bl + A1_br, A1_blt + A1_brt], axis=1)
    w2l = jnp.concatenate([A2_Wl.transpose(0, 2, 1),
                           A2_Wlt.transpose(0, 2, 1)], axis=2)
    w2r = jnp.concatenate([A2_Wr.transpose(0, 2, 1),
                           A2_Wrt.transpose(0, 2, 1)], axis=2)
    b2 = jnp.concatenate([A2_bl + A2_br, A2_blt + A2_brt], axis=1)

    pw = jnp.stack([pw1[:, 0], pw2[:, 0], pw1[:, 1],
                    pw1[:, 2], pw2[:, 1], pw3[:, 0]], axis=0)
    mtg = MTG_W.T * (1.0 / 3.0)
    cb = ((pb1 + pb2 + pb3) * (1.0 / 3.0)) @ MTG_W.T + MTG_b

    return _dense(h0, h1, n2s, w1l, w1r, b1, w2l, w2r, b2, pw, mtg, cb)


# speedup vs baseline: 1.0007x; 1.0007x over previous
import functools

import jax
import jax.numpy as jnp
from jax import lax
from jax.experimental import pallas as pl
from jax.experimental.pallas import tpu as pltpu
from jax.experimental.pallas import tpu_sc as plsc

B = 10000
N = 100000
D = 128
H1 = 128
H2 = 64
C = 64
T = 3
S0, S1 = 5, 2

_NC, _NS = 2, 16
_NW = _NC * _NS

BP = 10240
BS = 400
G = B // BS

STEPS = [(0, 0, 0), (0, 1, 0), (1, 0, 1), (2, 0, 2), (2, 1, 1), (2, 2, 0)]


def _mk_gather(rows, nch, ch, nb):
    per_w = nch * ch
    assert rows == _NW * per_w and nch % nb == 0
    mesh = plsc.VectorSubcoreMesh(core_axis_name="c", subcore_axis_name="s")

    @functools.partial(
        pl.kernel,
        out_type=jax.ShapeDtypeStruct((rows, D), jnp.float32),
        mesh=mesh,
        scratch_types=(
            [pltpu.VMEM((nch, ch), jnp.int32)]
            + [pltpu.VMEM((ch, D), jnp.float32) for _ in range(nb)]
            + [pltpu.SemaphoreType.DMA for _ in range(2 * nb)]
        ),
    )
    def k(x_hbm, idx_hbm, out_hbm, idx_v, *rest):
        bufs = rest[:nb]
        sems = rest[nb:2 * nb]
        wsems = rest[2 * nb:]
        wid = lax.axis_index("s") * _NC + lax.axis_index("c")
        base = wid * per_w
        pltpu.sync_copy(idx_hbm.at[wid], idx_v)

        @pl.loop(0, nch // nb)
        def _grp(i):
            j0 = i * nb
            cps = [None] * nb
            for b in range(nb):
                @pl.when(i > 0)
                def _(b=b):
                    pltpu.make_async_copy(
                        bufs[b], out_hbm.at[pl.ds(0, ch)], wsems[b]).wait()
                cps[b] = pltpu.async_copy(x_hbm.at[idx_v.at[j0 + b]],
                                          bufs[b], sems[b])
            for b in range(nb):
                cps[b].wait()
                pltpu.async_copy(
                    bufs[b], out_hbm.at[pl.ds(base + (j0 + b) * ch, ch)],
                    wsems[b])
        for b in range(nb):
            pltpu.make_async_copy(
                bufs[b], out_hbm.at[pl.ds(0, ch)], wsems[b]).wait()

    return k


def _mk_gather_pairsum(rows, nch, ch, nb):
    per_w = nch * ch
    assert rows == _NW * per_w and nch % nb == 0
    mesh = plsc.VectorSubcoreMesh(core_axis_name="c", subcore_axis_name="s")

    @functools.partial(
        pl.kernel,
        out_type=jax.ShapeDtypeStruct((rows, D), jnp.float32),
        mesh=mesh,
        scratch_types=(
            [pltpu.VMEM((nch, ch), jnp.int32) for _ in range(2)]
            + [pltpu.VMEM((ch, D), jnp.float32) for _ in range(nb)]
            + [pltpu.SemaphoreType.DMA for _ in range(3 * nb)]
        ),
    )
    def k(x_hbm, idxe_hbm, idxo_hbm, out_hbm, idxe_v, idxo_v, *rest):
        bufs = rest[:nb]
        esems = rest[nb:2 * nb]
        osems = rest[2 * nb:3 * nb]
        wsems = rest[3 * nb:]
        wid = lax.axis_index("s") * _NC + lax.axis_index("c")
        base = wid * per_w
        pltpu.sync_copy(idxe_hbm.at[wid], idxe_v)
        pltpu.sync_copy(idxo_hbm.at[wid], idxo_v)

        @pl.loop(0, nch // nb)
        def _grp(i):
            j0 = i * nb
            ecps = [None] * nb
            for b in range(nb):
                @pl.when(i > 0)
                def _(b=b):
                    pltpu.make_async_copy(
                        bufs[b], out_hbm.at[pl.ds(0, ch)], wsems[b]).wait()
                ecps[b] = pltpu.async_copy(x_hbm.at[idxe_v.at[j0 + b]],
                                           bufs[b], esems[b])
            ocps = [None] * nb
            for b in range(nb):
                ecps[b].wait()
                ocps[b] = pltpu.async_copy(x_hbm.at[idxo_v.at[j0 + b]],
                                           bufs[b], osems[b], add=True)
            for b in range(nb):
                ocps[b].wait()
                pltpu.async_copy(
                    bufs[b], out_hbm.at[pl.ds(base + (j0 + b) * ch, ch)],
                    wsems[b])
        for b in range(nb):
            pltpu.make_async_copy(
                bufs[b], out_hbm.at[pl.ds(0, ch)], wsems[b]).wait()

    return k


_gather_h0 = _mk_gather(BP, 4, 80, 4)
_gather_h1 = _mk_gather(T * S0 * BP, 40, 120, 8)
_gather_n2 = _mk_gather_pairsum(T * S0 * BP, 40, 120, 5)


def _dense_body(h0_ref, h1_ref, n2_ref,
                w1l_ref, w1r_ref, b1_ref,
                w2l_ref, w2r_ref, b2_ref,
                pw_ref, mtg_ref, cb_ref, out_ref):
    f32 = jnp.float32
    bf16 = jnp.bfloat16
    h0 = h0_ref[...].astype(bf16)
    acc = jnp.zeros((BS, C), f32)
    v1 = {}
    v2 = {}
    feats = {}

    def get_feats(t):
        if t not in feats:
            h1 = h1_ref[t]
            n1 = (h1[0] + h1[1] + h1[2] + h1[3] + h1[4]) * (1.0 / S0)
            n2 = n2_ref[t] * (1.0 / S1)
            xcat = jnp.concatenate(
                [h0, h1.astype(bf16).reshape(S0 * BS, D)], axis=0)
            ncat = jnp.concatenate(
                [n1.astype(bf16), n2.astype(bf16).reshape(S0 * BS, D)],
                axis=0)
            feats[t] = (xcat, ncat)
        return feats[t]

    for si, (t, c, _k) in enumerate(STEPS):
        xcat, ncat = get_feats(t)
        u = (jnp.dot(xcat, w1l_ref[c], preferred_element_type=f32)
             + jnp.dot(ncat, w1r_ref[c], preferred_element_type=f32)
             + b1_ref[c][None, :])
        ux = u[:, :H1]
        ut = u[:, H1:]
        v = v1.get(c, 0.0) + ut
        spk = (v >= 1.0).astype(f32)
        v1[c] = v * (1.0 - spk)
        outs = (jax.nn.sigmoid(ux) * spk).astype(bf16)
        h0n = outs[:BS]
        nb_ = outs[BS:].reshape(S0, BS, H1).astype(f32)
        nb_ = ((nb_[0] + nb_[1] + nb_[2] + nb_[3] + nb_[4])
               * (1.0 / S0)).astype(bf16)
        u2 = (jnp.dot(h0n, w2l_ref[c], preferred_element_type=f32)
              + jnp.dot(nb_, w2r_ref[c], preferred_element_type=f32)
              + b2_ref[c][None, :])
        ux2 = u2[:, :H2]
        ut2 = u2[:, H2:]
        v_ = v2.get(c, 0.0) + ut2
        spk2 = (v_ >= 1.0).astype(f32)
        v2[c] = v_ * (1.0 - spk2)
        o = jax.nn.sigmoid(ux2) * spk2
        acc = acc + o * pw_ref[si][None, :]
    out_ref[...] = jnp.dot(acc, mtg_ref[...], preferred_element_type=f32) \
        + cb_ref[...][None, :]


def _dense(h0, h1, n2s, w1l, w1r, b1, w2l, w2r, b2, pw, mtg, cb):
    return pl.pallas_call(
        _dense_body,
        grid=(G,),
        in_specs=[
            pl.BlockSpec((BS, D), lambda i: (i, 0)),
            pl.BlockSpec((T, S0, BS, D), lambda i: (0, 0, i, 0)),
            pl.BlockSpec((T, S0, BS, D), lambda i: (0, 0, i, 0)),
            pl.BlockSpec((T, D, 2 * H1), lambda i: (0, 0, 0)),
            pl.BlockSpec((T, D, 2 * H1), lambda i: (0, 0, 0)),
            pl.BlockSpec((T, 2 * H1), lambda i: (0, 0)),
            pl.BlockSpec((T, H1, 2 * H2), lambda i: (0, 0, 0)),
            pl.BlockSpec((T, H1, 2 * H2), lambda i: (0, 0, 0)),
            pl.BlockSpec((T, 2 * H2), lambda i: (0, 0)),
            pl.BlockSpec((len(STEPS), H2), lambda i: (0, 0)),
            pl.BlockSpec((H2, C), lambda i: (0, 0)),
            pl.BlockSpec((C,), lambda i: (0,)),
        ],
        out_specs=pl.BlockSpec((BS, C), lambda i: (i, 0)),
        out_shape=jax.ShapeDtypeStruct((B, C), jnp.float32),
        compiler_params=pltpu.CompilerParams(
            dimension_semantics=("arbitrary",),
        ),
    )(h0, h1, n2s, w1l, w1r, b1, w2l, w2r, b2, pw, mtg, cb)


def kernel(x, nodes, nbr1, nbr2,
           A1_Wl, A1_Wr, A1_Wlt, A1_Wrt, A1_bl, A1_br, A1_blt, A1_brt,
           A2_Wl, A2_Wr, A2_Wlt, A2_Wrt, A2_bl, A2_br, A2_blt, A2_brt,
           MTG_W, MTG_b, pw1, pb1, pw2, pb2, pw3, pb3):
    nodes = nodes.astype(jnp.int32)
    nbr1 = nbr1.astype(jnp.int32)
    nbr2 = nbr2.astype(jnp.int32)
    pad = BP - B

    idx0 = jnp.pad(nodes, (0, pad)).reshape(_NW, 4, 80)
    idx1 = nbr1.reshape(T, B, S0).transpose(0, 2, 1)
    idx1 = jnp.pad(idx1, ((0, 0), (0, 0), (0, pad))).reshape(_NW, 40, 120)
    idx2 = nbr2.reshape(T, B, S0, S1).transpose(0, 2, 1, 3)
    idxe = jnp.pad(idx2[..., 0], ((0, 0), (0, 0), (0, pad))).reshape(_NW, 40, 120)
    idxo = jnp.pad(idx2[..., 1], ((0, 0), (0, 0), (0, pad))).reshape(_NW, 40, 120)

    h0 = _gather_h0(x, idx0)
    h1 = _gather_h1(x, idx1).reshape(T, S0, BP, D)
    n2s = _gather_n2(x, idxe, idxo).reshape(T, S0, BP, D)

    bf16 = jnp.bfloat16
    w1l = jnp.concatenate([A1_Wl.transpose(0, 2, 1),
                           A1_Wlt.transpose(0, 2, 1)], axis=2).astype(bf16)
    w1r = jnp.concatenate([A1_Wr.transpose(0, 2, 1),
                           A1_Wrt.transpose(0, 2, 1)], axis=2).astype(bf16)
    b1 = jnp.concatenate([A1_bl + A1_br, A1_blt + A1_brt], axis=1)
    w2l = jnp.concatenate([A2_Wl.transpose(0, 2, 1),
                           A2_Wlt.transpose(0, 2, 1)], axis=2).astype(bf16)
    w2r = jnp.concatenate([A2_Wr.transpose(0, 2, 1),
                           A2_Wrt.transpose(0, 2, 1)], axis=2).astype(bf16)
    b2 = jnp.concatenate([A2_bl + A2_br, A2_blt + A2_brt], axis=1)

    pw = jnp.stack([pw1[:, 0], pw2[:, 0], pw1[:, 1],
                    pw1[:, 2], pw2[:, 1], pw3[:, 0]], axis=0)
    mtg = MTG_W.T * (1.0 / 3.0)
    cb = ((pb1 + pb2 + pb3) * (1.0 / 3.0)) @ MTG_W.T + MTG_b

    return _dense(h0, h1, n2s, w1l, w1r, b1, w2l, w2r, b2, pw, mtg, cb)
